# baseline (speedup 1.0000x reference)
import jax
import jax.numpy as jnp
from jax.experimental import pallas as pl
from jax.experimental.pallas import tpu as pltpu

_B, _C, _H, _W = 8, 19, 512, 512
_NEIGH_W = 3
_EPS = 1e-12
_CH = 128
_CPB = _H // _CH
_NCHUNK = _B * _CPB
_NBUF = 4
_AHEAD = 2


def _argmax_c(x):
    m = x[0]
    idx = jnp.zeros(x.shape[1:], dtype=jnp.int32)
    for c in range(1, x.shape[0]):
        pred = x[c] > m
        m = jnp.where(pred, x[c], m)
        idx = jnp.where(pred, c, idx)
    return idx


def _compute(s, bufs, rowbufs, carry_ref, inv_table_ref):
    slot = s % _NBUF
    x = bufs[slot]
    lab = _argmax_c(x)

    minus1 = jnp.full((1, _W), -1, dtype=jnp.int32)
    if s % _CPB == 0:
        lab_top = minus1
    else:
        lab_top = carry_ref[0:1, :]
    if s % _CPB == _CPB - 1:
        lab_bot = minus1
    else:
        lab_bot = _argmax_c(rowbufs[(s + 1) % _NBUF])

    L = jnp.concatenate([lab_top, lab, lab_bot], axis=0)
    carry_ref[0:1, :] = lab[_CH - 1 : _CH, :]

    count = jnp.zeros(lab.shape, dtype=jnp.int32)
    mcol = jnp.full((_CH, 1), -1, dtype=jnp.int32)
    for di in range(3):
        rows = L[di : di + _CH, :]
        for dj in range(3):
            if dj == 0:
                sh = jnp.concatenate([mcol, rows[:, : _W - 1]], axis=1)
            elif dj == 2:
                sh = jnp.concatenate([rows[:, 1:], mcol], axis=1)
            else:
                sh = rows
            count = count + (sh == lab).astype(jnp.int32)
    n = count - 1

    inv = jnp.full(lab.shape, inv_table_ref[0], dtype=jnp.float32)
    for k in range(1, _NEIGH_W * _NEIGH_W):
        inv = jnp.where(n == k, inv_table_ref[k], inv)

    bufs[slot] = x * inv[None, :, :]


def _nectar_kernel(inv_table_ref, x, o, bufs, rowbufs, carry_ref, rsems, wsems, rowsems):
    def read(s):
        b, r0 = s // _CPB, (s % _CPB) * _CH
        return pltpu.make_async_copy(
            x.at[b, :, pl.ds(r0, _CH), :], bufs.at[s % _NBUF], rsems.at[s % _NBUF]
        )

    def rowread(s):
        b, r0 = s // _CPB, (s % _CPB) * _CH
        return pltpu.make_async_copy(
            x.at[b, :, pl.ds(r0, 1), :], rowbufs.at[s % _NBUF], rowsems.at[s % _NBUF]
        )

    def write(s):
        b, r0 = s // _CPB, (s % _CPB) * _CH
        return pltpu.make_async_copy(
            bufs.at[s % _NBUF], o.at[b, :, pl.ds(r0, _CH), :], wsems.at[s % _NBUF]
        )

    for s in range(_AHEAD):
        read(s).start()
    for s in range(1, _AHEAD):
        if s < _NCHUNK and s % _CPB:
            rowread(s).start()
    for s in range(_NCHUNK):
        read(s).wait()
        if s + 1 < _NCHUNK and (s + 1) % _CPB:
            rowread(s + 1).wait()
        nxt = s + _AHEAD
        if nxt < _NCHUNK:
            prev = nxt - _NBUF
            if prev >= 0:
                write(prev).wait()
            read(nxt).start()
            if nxt % _CPB:
                rowread(nxt).start()
        _compute(s, bufs, rowbufs, carry_ref, inv_table_ref)
        write(s).start()
    for s in range(_NCHUNK - _NBUF, _NCHUNK):
        write(s).wait()


@jax.jit
def kernel(logits, neighborhood_temps):
    inv_table = 1.0 / (jax.nn.relu(neighborhood_temps) + _EPS)
    return pl.pallas_call(
        _nectar_kernel,
        in_specs=[
            pl.BlockSpec(memory_space=pltpu.MemorySpace.SMEM),
            pl.BlockSpec(memory_space=pltpu.MemorySpace.HBM),
        ],
        out_specs=pl.BlockSpec(memory_space=pltpu.MemorySpace.HBM),
        scratch_shapes=[
            pltpu.VMEM((_NBUF, _C, _CH, _W), jnp.float32),
            pltpu.VMEM((_NBUF, _C, 1, _W), jnp.float32),
            pltpu.VMEM((8, _W), jnp.int32),
            pltpu.SemaphoreType.DMA((_NBUF,)),
            pltpu.SemaphoreType.DMA((_NBUF,)),
            pltpu.SemaphoreType.DMA((_NBUF,)),
        ],
        out_shape=jax.ShapeDtypeStruct(logits.shape, logits.dtype),
    )(inv_table, logits)


# speedup vs baseline: 1.0378x; 1.0378x over previous
import jax
import jax.numpy as jnp
from jax.experimental import pallas as pl
from jax.experimental.pallas import tpu as pltpu

_B, _C, _H, _W = 8, 19, 512, 512
_NEIGH_W = 3
_EPS = 1e-12
_HB = 256
_HALO = 8


def _argmax_c(x):
    m = x[0]
    idx = jnp.zeros(x.shape[1:], dtype=jnp.int32)
    for c in range(1, x.shape[0]):
        pred = x[c] > m
        m = jnp.where(pred, x[c], m)
        idx = jnp.where(pred, c, idx)
    return idx


def _nectar_kernel(inv_table_ref, logits_ref, bot_ref, out_ref, carry_ref):
    i = pl.program_id(1)
    n_i = pl.num_programs(1)

    x = logits_ref[0]
    lab = _argmax_c(x)

    minus1 = jnp.full((1, _W), -1, dtype=jnp.int32)
    lab_top = jnp.where(i == 0, minus1, carry_ref[0:1, :])
    lab_bot = _argmax_c(bot_ref[0, :, 0:1, :])
    lab_bot = jnp.where(i == n_i - 1, minus1, lab_bot)

    L = jnp.concatenate([lab_top, lab, lab_bot], axis=0)
    carry_ref[0:1, :] = lab[_HB - 1 : _HB, :]

    count = jnp.zeros(lab.shape, dtype=jnp.int32)
    mcol = jnp.full((_HB, 1), -1, dtype=jnp.int32)
    for di in range(3):
        rows = L[di : di + _HB, :]
        for dj in range(3):
            if dj == 0:
                s = jnp.concatenate([mcol, rows[:, : _W - 1]], axis=1)
            elif dj == 2:
                s = jnp.concatenate([rows[:, 1:], mcol], axis=1)
            else:
                s = rows
            count = count + (s == lab).astype(jnp.int32)
    n = count - 1

    inv = jnp.full(lab.shape, inv_table_ref[0], dtype=jnp.float32)
    for k in range(1, _NEIGH_W * _NEIGH_W):
        inv = jnp.where(n == k, inv_table_ref[k], inv)

    out_ref[0] = x * inv[None, :, :]


@jax.jit
def kernel(logits, neighborhood_temps):
    inv_table = 1.0 / (jax.nn.relu(neighborhood_temps) + _EPS)
    n_i = _H // _HB

    grid = (_B, n_i)
    in_specs = [
        pl.BlockSpec((1, _C, _HB, _W), lambda b, i, *_: (b, 0, i, 0)),
        pl.BlockSpec(
            (1, _C, _HALO, _W),
            lambda b, i, *_: (
                b,
                0,
                jnp.minimum((i + 1) * (_HB // _HALO), _H // _HALO - 1),
                0,
            ),
        ),
    ]
    out_spec = pl.BlockSpec((1, _C, _HB, _W), lambda b, i, *_: (b, 0, i, 0))

    return pl.pallas_call(
        _nectar_kernel,
        grid_spec=pltpu.PrefetchScalarGridSpec(
            num_scalar_prefetch=1,
            grid=grid,
            in_specs=in_specs,
            out_specs=out_spec,
            scratch_shapes=[pltpu.VMEM((8, _W), jnp.int32)],
        ),
        out_shape=jax.ShapeDtypeStruct(logits.shape, logits.dtype),
        compiler_params=pltpu.CompilerParams(
            dimension_semantics=("parallel", "arbitrary")
        ),
    )(inv_table, logits, logits)
